# Initial kernel scaffold; baseline (speedup 1.0000x reference)
import jax
import jax.numpy as jnp
from jax.experimental import pallas as pl


def kernel(input_data, emb_table, W, b):
    raise NotImplementedError("write your pallas kernel here")


# speedup vs baseline: 25.4855x; 25.4855x over previous
import functools

import jax
import jax.numpy as jnp
from jax import lax
from jax.experimental import pallas as pl
from jax.experimental.pallas import tpu as pltpu
from jax.experimental.pallas import tpu_sc as plsc

_VOCAB = 100000
_EMBED = 100
_SEQ = 200
_BATCH = 4096
_NCLS = 2

_VB = 4096
_NW = 32
_SPW = (_SEQ + _NW - 1) // _NW
_LANES = 16


def _mm_body(w0_ref, w1_ref, tab_ref, p_ref):
    wd = w0_ref[...] - w1_ref[...]
    p_ref[...] = lax.dot_general(
        wd, tab_ref[...],
        dimension_numbers=(((1,), (1,)), ((), ())),
        preferred_element_type=jnp.float32,
    )


def _make_p(w0, w1, table):
    grid = (_VOCAB + _VB - 1) // _VB
    return pl.pallas_call(
        _mm_body,
        grid=(grid,),
        in_specs=[
            pl.BlockSpec((_SEQ, _EMBED), lambda i: (0, 0)),
            pl.BlockSpec((_SEQ, _EMBED), lambda i: (0, 0)),
            pl.BlockSpec((_VB, _EMBED), lambda i: (i, 0)),
        ],
        out_specs=pl.BlockSpec((_SEQ, _VB), lambda i: (0, i)),
        out_shape=jax.ShapeDtypeStruct((_SEQ, _VOCAB), jnp.float32),
    )(w0, w1, table)


def _sc_gather_body(p_hbm, idxt_hbm, out_hbm, row_v, idx_v, acc_v):
    wid = lax.axis_index("s") * 2 + lax.axis_index("c")

    def zero_body(i, carry):
        acc_v[pl.ds(i * _LANES, _LANES)] = jnp.zeros((_LANES,), jnp.float32)
        return carry

    lax.fori_loop(0, _BATCH // _LANES, zero_body, 0, unroll=8)

    def s_body(j, carry):
        sidx = wid + _NW * j

        @pl.when(sidx < _SEQ)
        def _():
            pltpu.sync_copy(idxt_hbm.at[sidx], idx_v)
            pltpu.sync_copy(p_hbm.at[sidx], row_v)

            def g_body(i, c):
                iv = idx_v[pl.ds(i * _LANES, _LANES)]
                vals = plsc.load_gather(row_v, [iv])
                acc_v[pl.ds(i * _LANES, _LANES)] = (
                    acc_v[pl.ds(i * _LANES, _LANES)] + vals)
                return c

            lax.fori_loop(0, _BATCH // _LANES, g_body, 0, unroll=8)

        return carry

    lax.fori_loop(0, _SPW, s_body, 0)
    pltpu.sync_copy(acc_v, out_hbm.at[wid])


def _sc_gather(p, idxt):
    mesh = plsc.VectorSubcoreMesh(core_axis_name="c", subcore_axis_name="s")
    kfn = functools.partial(
        pl.kernel,
        mesh=mesh,
        compiler_params=pltpu.CompilerParams(needs_layout_passes=False),
        out_type=jax.ShapeDtypeStruct((_NW, _BATCH), jnp.float32),
        scratch_types=[
            pltpu.VMEM((_VOCAB,), jnp.float32),
            pltpu.VMEM((_BATCH,), jnp.int32),
            pltpu.VMEM((_BATCH,), jnp.float32),
        ],
    )(_sc_gather_body)
    return kfn(p, idxt)


def _fin_body(part_ref, bias_ref, out_ref):
    d = jnp.sum(part_ref[...], axis=0, keepdims=True)
    bd = bias_ref[...][0:1, 0:1] - bias_ref[...][0:1, 1:2]
    d = d + bd
    ad = jnp.abs(d)
    t = jnp.log1p(jnp.exp(-ad))
    sp_pos = jnp.maximum(d, 0.0) + t
    sp_neg = jnp.maximum(-d, 0.0) + t
    out_ref[...] = jnp.concatenate([-sp_neg, -sp_pos], axis=0)


def _finalize(partials, b):
    return pl.pallas_call(
        _fin_body,
        out_shape=jax.ShapeDtypeStruct((_NCLS, _BATCH), jnp.float32),
    )(partials, b.reshape(1, _NCLS).astype(jnp.float32))


def kernel(input_data, emb_table, W, b):
    idx = input_data.astype(jnp.int32)
    idxt = idx.T
    wr = W.reshape(_SEQ, _EMBED, _NCLS)
    w0 = wr[:, :, 0]
    w1 = wr[:, :, 1]
    p = _make_p(w0, w1, emb_table)
    partials = _sc_gather(p, idxt)
    out2 = _finalize(partials, b)
    return out2.T
